# Initial kernel scaffold; baseline (speedup 1.0000x reference)
import jax
import jax.numpy as jnp
from jax.experimental import pallas as pl


def kernel(x, adj_indices, adj_values):
    raise NotImplementedError("write your pallas kernel here")


# speedup vs baseline: 6.5355x; 6.5355x over previous
import functools

import jax
import jax.numpy as jnp
from jax import lax
from jax.experimental import pallas as pl
from jax.experimental.pallas import tpu as pltpu
from jax.experimental.pallas import tpu_sc as plsc

N_NODES = 10000
N_EDGES = 320000
D_FEAT = 128

NC = 2
NS = 16
LANES = 16
NW = NC * NS
EPW = N_EDGES // NW
CHUNK = 80
NCHUNK = EPW // CHUNK
GROUPS = CHUNK // LANES
SB = 25
NSUPER = NCHUNK // SB
RPT = 624
RTAIL = N_NODES - NS * RPT


def _sc_body(x_hbm, row_hbm, col_hbm, val_hbm, z_hbm, out_hbm,
             row_v, col_v, val_v, gbuf, acc_sh, sem):
    cid = lax.axis_index("c")
    sid = lax.axis_index("s")
    wid = sid * NC + cid

    pltpu.sync_copy(z_hbm.at[pl.ds(sid * RPT, RPT)],
                    acc_sh.at[pl.ds(sid * RPT, RPT)])

    @pl.when(sid == 0)
    def _():
        pltpu.sync_copy(z_hbm.at[pl.ds(NS * RPT, RTAIL)],
                        acc_sh.at[pl.ds(NS * RPT, RTAIL)])

    plsc.subcore_barrier()

    def super_body(s_i, carry0):
        pltpu.sync_copy(row_hbm.at[wid, s_i], row_v)
        pltpu.sync_copy(col_hbm.at[wid, s_i], col_v)
        pltpu.sync_copy(val_hbm.at[wid, s_i], val_v)

        def chunk_body(c, carry):
            pltpu.async_copy(x_hbm.at[col_v.at[c]], gbuf, sem).wait()

            def group_body(g, carry2):
                v16 = val_v[c, pl.ds(g * LANES, LANES)]
                for i in range(LANES):
                    s = v16.at[jnp.full((LANES,), i, jnp.int32)].get(
                        mode="promise_in_bounds")
                    e = g * LANES + i
                    for f in range(D_FEAT // LANES):
                        sl = pl.ds(f * LANES, LANES)
                        gbuf[e, sl] = gbuf[e, sl] * s
                return carry2

            lax.fori_loop(0, GROUPS, group_body, 0)

            pltpu.sync_copy(gbuf, acc_sh.at[row_v.at[c]], add=True)
            return carry

        lax.fori_loop(0, SB, chunk_body, 0)
        return carry0

    lax.fori_loop(0, NSUPER, super_body, 0)
    plsc.subcore_barrier()

    pltpu.sync_copy(acc_sh.at[pl.ds(sid * RPT, RPT)],
                    out_hbm.at[cid, pl.ds(sid * RPT, RPT)])

    @pl.when(sid == 0)
    def _():
        pltpu.sync_copy(acc_sh.at[pl.ds(NS * RPT, RTAIL)],
                        out_hbm.at[cid, pl.ds(NS * RPT, RTAIL)])


def _combine_body(p_ref, o_ref):
    o_ref[...] = p_ref[0] + p_ref[1]


@jax.jit
def kernel(x, adj_indices, adj_values):
    idx = adj_indices.astype(jnp.int32)
    row3 = idx[0].reshape(NW, NSUPER, SB, CHUNK)
    col3 = idx[1].reshape(NW, NSUPER, SB, CHUNK)
    val3 = adj_values.reshape(NW, NSUPER, SB, CHUNK)
    zeros = jnp.zeros((N_NODES, D_FEAT), jnp.float32)

    mesh = plsc.VectorSubcoreMesh(core_axis_name="c", subcore_axis_name="s",
                                  num_cores=NC, num_subcores=NS)
    partials = pl.kernel(
        _sc_body,
        out_type=jax.ShapeDtypeStruct((NC, N_NODES, D_FEAT), jnp.float32),
        mesh=mesh,
        scratch_types=[
            pltpu.VMEM((SB, CHUNK), jnp.int32),
            pltpu.VMEM((SB, CHUNK), jnp.int32),
            pltpu.VMEM((SB, CHUNK), jnp.float32),
            pltpu.VMEM((CHUNK, D_FEAT), jnp.float32),
            pltpu.VMEM_SHARED((N_NODES, D_FEAT), jnp.float32),
            pltpu.SemaphoreType.DMA,
        ],
    )(x, row3, col3, val3, zeros)

    blk = 1000
    return pl.pallas_call(
        _combine_body,
        out_shape=jax.ShapeDtypeStruct((N_NODES, D_FEAT), jnp.float32),
        grid=(N_NODES // blk,),
        in_specs=[pl.BlockSpec((NC, blk, D_FEAT), lambda i: (0, i, 0))],
        out_specs=pl.BlockSpec((blk, D_FEAT), lambda i: (i, 0)),
    )(partials)
